# baseline (speedup 1.0000x reference)
import functools

import jax
import jax.numpy as jnp
from jax import lax
from jax.experimental import pallas as pl
from jax.experimental.pallas import tpu as pltpu
from jax.experimental.pallas import tpu_sc as plsc


def _linear_body(x_ref, w_ref, b_ref, o_ref):
    o_ref[...] = (
        jnp.dot(x_ref[...], w_ref[...], preferred_element_type=jnp.float32)
        + b_ref[...]
    )


def _linear(x, w, b, block_rows=None):
    m, k = x.shape
    _, n = w.shape
    if block_rows is None:
        block_rows = m
    grid = (m // block_rows,)
    return pl.pallas_call(
        _linear_body,
        grid=grid,
        in_specs=[
            pl.BlockSpec((block_rows, k), lambda i: (i, 0)),
            pl.BlockSpec((k, n), lambda i: (0, 0)),
            pl.BlockSpec((1, n), lambda i: (0, 0)),
        ],
        out_specs=pl.BlockSpec((block_rows, n), lambda i: (i, 0)),
        out_shape=jax.ShapeDtypeStruct((m, n), jnp.float32),
    )(x, w, b.reshape(1, -1))


def _edge_math(xj, xi, e, wxi_ref, wxj_ref, wee_ref, bg_ref, bn_ref, be_ref,
               msg_ref, eout_ref):
    h = e.shape[1]
    p_i = jnp.dot(xi, wxi_ref[...], preferred_element_type=jnp.float32)
    p_j = jnp.dot(xj, wxj_ref[...], preferred_element_type=jnp.float32)
    p_e = jnp.dot(e, wee_ref[...], preferred_element_type=jnp.float32)
    gate = jax.nn.sigmoid(p_i[:, :h] + p_e[:, :h] + bg_ref[...])
    msg_ref[...] = gate * (p_j[:, :h] + p_e[:, h:2 * h] + bn_ref[...])
    newe = p_i[:, h:] + p_j[:, h:] + p_e[:, 2 * h:] + be_ref[...]
    eout_ref[...] = (e + jnp.maximum(newe, 0.0)).astype(jnp.bfloat16)


def _edge_body(xj_ref, xi_ref, e_ref, wxi_ref, wxj_ref, wee_ref, bg_ref,
               bn_ref, be_ref, msg_ref, eout_ref):
    _edge_math(xj_ref[...], xi_ref[...],
               e_ref[...].astype(jnp.float32), wxi_ref, wxj_ref, wee_ref,
               bg_ref, bn_ref, be_ref, msg_ref, eout_ref)


def _edge_first_body(xj_ref, xi_ref, ea_ref, wea_ref, bea_ref, wxi_ref,
                     wxj_ref, wee_ref, bg_ref, bn_ref, be_ref, msg_ref,
                     eout_ref):
    e = (jnp.dot(ea_ref[...], wea_ref[...], preferred_element_type=jnp.float32)
         + bea_ref[...])
    _edge_math(xj_ref[...], xi_ref[...], e,
               wxi_ref, wxj_ref, wee_ref, bg_ref, bn_ref, be_ref, msg_ref,
               eout_ref)


def _edge_layer(gath, e, wxi, wxj, wee, bg, bn, be, n_edges, block_edges):
    hdim = wxi.shape[0]
    nblk = n_edges // block_edges
    first = isinstance(e, tuple)
    if first:
        e, wea, bea = e
        d_e = e.shape[1]
        extra_specs = [
            pl.BlockSpec((block_edges, d_e), lambda i: (i, 0)),
            pl.BlockSpec((d_e, hdim), lambda i: (0, 0)),
            pl.BlockSpec((1, hdim), lambda i: (0, 0)),
        ]
        extra_args = (e, wea, bea.reshape(1, -1))
        body = _edge_first_body
    else:
        extra_specs = [pl.BlockSpec((block_edges, hdim), lambda i: (i, 0))]
        extra_args = (e,)
        body = _edge_body
    return pl.pallas_call(
        body,
        grid=(nblk,),
        in_specs=[
            pl.BlockSpec((block_edges, hdim), lambda i: (i, 0)),
            pl.BlockSpec((block_edges, hdim), lambda i: (i + nblk, 0)),
            *extra_specs,
            pl.BlockSpec((hdim, 2 * hdim), lambda i: (0, 0)),
            pl.BlockSpec((hdim, 2 * hdim), lambda i: (0, 0)),
            pl.BlockSpec((hdim, 3 * hdim), lambda i: (0, 0)),
            pl.BlockSpec((1, hdim), lambda i: (0, 0)),
            pl.BlockSpec((1, hdim), lambda i: (0, 0)),
            pl.BlockSpec((1, hdim), lambda i: (0, 0)),
        ],
        out_specs=[
            pl.BlockSpec((block_edges, hdim), lambda i: (i, 0)),
            pl.BlockSpec((block_edges, hdim), lambda i: (i, 0)),
        ],
        out_shape=[
            jax.ShapeDtypeStruct((n_edges, hdim), jnp.float32),
            jax.ShapeDtypeStruct((n_edges, hdim), jnp.bfloat16),
        ],
    )(gath, gath, *extra_args, wxi, wxj, wee, bg.reshape(1, -1),
      bn.reshape(1, -1), be.reshape(1, -1))


def _edge_last_body(xj_ref, xi_ref, e_ref, wgx_ref, wnx_ref, wge_ref, bg_ref,
                    bn_ref, msg_ref):
    h = e_ref.shape[1]
    p_i = jnp.dot(xi_ref[...], wgx_ref[...], preferred_element_type=jnp.float32)
    p_j = jnp.dot(xj_ref[...], wnx_ref[...], preferred_element_type=jnp.float32)
    p_e = jnp.dot(e_ref[...].astype(jnp.float32), wge_ref[...],
                  preferred_element_type=jnp.float32)
    gate = jax.nn.sigmoid(p_i + p_e[:, :h] + bg_ref[...])
    msg_ref[...] = gate * (p_j + p_e[:, h:] + bn_ref[...])


def _edge_layer_last(gath, e, wgx, wnx, wge, bg, bn, n_edges, block_edges):
    hdim = e.shape[1]
    nblk = n_edges // block_edges
    return pl.pallas_call(
        _edge_last_body,
        grid=(nblk,),
        in_specs=[
            pl.BlockSpec((block_edges, hdim), lambda i: (i, 0)),
            pl.BlockSpec((block_edges, hdim), lambda i: (i + nblk, 0)),
            pl.BlockSpec((block_edges, hdim), lambda i: (i, 0)),
            pl.BlockSpec((hdim, hdim), lambda i: (0, 0)),
            pl.BlockSpec((hdim, hdim), lambda i: (0, 0)),
            pl.BlockSpec((hdim, 2 * hdim), lambda i: (0, 0)),
            pl.BlockSpec((1, hdim), lambda i: (0, 0)),
            pl.BlockSpec((1, hdim), lambda i: (0, 0)),
        ],
        out_specs=pl.BlockSpec((block_edges, hdim), lambda i: (i, 0)),
        out_shape=jax.ShapeDtypeStruct((n_edges, hdim), jnp.float32),
    )(gath, gath, e, wgx, wnx, wge, bg.reshape(1, -1), bn.reshape(1, -1))


def _node_update_body(h_ref, *refs):
    p_refs, (o_ref,) = refs[:-1], refs[-1:]
    acc = p_refs[0][...]
    for p in p_refs[1:]:
        acc = acc + p[...]
    o_ref[...] = h_ref[...] + jnp.maximum(acc, 0.0)


def _partial_specs(partials_list, n_pad, block_rows, hdim):
    p1_blk = n_pad // block_rows
    specs, args = [], []
    for p in partials_list:
        specs.append(pl.BlockSpec((block_rows, hdim), lambda i: (i, 0)))
        specs.append(
            pl.BlockSpec((block_rows, hdim), lambda i: (i + p1_blk, 0)))
        args.extend([p, p])
    return specs, args


def _node_update(h, partials_list, n_pad, block_rows=80):
    n, hdim = h.shape
    p_specs, p_args = _partial_specs(partials_list, n_pad, block_rows, hdim)
    return pl.pallas_call(
        _node_update_body,
        grid=(n // block_rows,),
        in_specs=[pl.BlockSpec((block_rows, hdim), lambda i: (i, 0)), *p_specs],
        out_specs=pl.BlockSpec((block_rows, hdim), lambda i: (i, 0)),
        out_shape=jax.ShapeDtypeStruct((n, hdim), jnp.float32),
    )(h, *p_args)


def _final_body(h_ref, *refs):
    *p_refs, w1_ref, b1_ref, w2_ref, b2_ref, o_ref = refs
    acc = p_refs[0][...]
    for p in p_refs[1:]:
        acc = acc + p[...]
    hn = h_ref[...] + jnp.maximum(acc, 0.0)
    t = jnp.maximum(
        jnp.dot(hn, w1_ref[...], preferred_element_type=jnp.float32)
        + b1_ref[...], 0.0)
    o_ref[...] = jnp.sum(t * w2_ref[...], axis=1, keepdims=True) + b2_ref[...]


def _final(h, partials_list, w1, b1, w2, b2, n_pad, block_rows=80):
    n, hdim = h.shape
    h2 = w1.shape[1]
    p_specs, p_args = _partial_specs(partials_list, n_pad, block_rows, hdim)
    return pl.pallas_call(
        _final_body,
        grid=(n // block_rows,),
        in_specs=[
            pl.BlockSpec((block_rows, hdim), lambda i: (i, 0)),
            *p_specs,
            pl.BlockSpec((hdim, h2), lambda i: (0, 0)),
            pl.BlockSpec((1, h2), lambda i: (0, 0)),
            pl.BlockSpec((1, h2), lambda i: (0, 0)),
            pl.BlockSpec((1, 1), lambda i: (0, 0)),
        ],
        out_specs=pl.BlockSpec((block_rows, 1), lambda i: (i, 0)),
        out_shape=jax.ShapeDtypeStruct((n, 1), jnp.float32),
    )(h, *p_args, w1, b1.reshape(1, -1), w2.reshape(1, -1),
      b2.reshape(1, 1))


_NC = 2
_NS = 16
_NW = _NC * _NS


def _make_sc_gather(n_idx, n_rows, hdim, k, nbuf, dtype=jnp.float32):
    ch = n_idx // _NW
    nsteps = ch // k
    assert ch % k == 0 and nsteps % nbuf == 0 and k % 16 == 0 and k <= 128
    mesh = plsc.VectorSubcoreMesh(core_axis_name="c", subcore_axis_name="s",
                                  num_cores=_NC, num_subcores=_NS)

    @functools.partial(
        pl.kernel,
        out_type=jax.ShapeDtypeStruct((n_idx, hdim), dtype),
        mesh=mesh,
        scratch_types=[
            pltpu.VMEM((nsteps, k), jnp.int32),
            pltpu.VMEM((nbuf, k, hdim), dtype),
            pltpu.SemaphoreType.DMA,
            pltpu.SemaphoreType.DMA,
        ],
    )
    def gather_kernel(table_hbm, idx_hbm, out_hbm, idx_v, bufs, gsem, ssem):
        c = lax.axis_index("c")
        s = lax.axis_index("s")
        w = c * _NS + s
        base = w * ch
        pltpu.sync_copy(idx_hbm.at[w], idx_v)
        for b in range(nbuf):
            pltpu.async_copy(table_hbm.at[idx_v.at[b]], bufs.at[b], gsem)

        def group(gi, carry):
            for b in range(nbuf):
                step = gi * nbuf + b
                pltpu.make_async_copy(
                    table_hbm.at[idx_v.at[0]], bufs.at[b], gsem).wait()
                pltpu.async_copy(
                    bufs.at[b], out_hbm.at[pl.ds(base + step * k, k)], ssem)
                pltpu.make_async_copy(
                    bufs.at[b], out_hbm.at[pl.ds(base, k)], ssem).wait()
                nstep = step + nbuf

                @pl.when(nstep < nsteps)
                def _():
                    pltpu.async_copy(
                        table_hbm.at[idx_v.at[nstep]], bufs.at[b], gsem)
            return carry

        lax.fori_loop(0, nsteps // nbuf, group, 0)

    return gather_kernel


def _make_sc_scatter(n_edges, n_nodes, hdim, k, nbuf):
    ec = n_edges // _NW
    nsteps = ec // k
    rz = n_nodes // _NS
    ngroups, ntail = divmod(nsteps, nbuf)
    assert ec % k == 0 and n_nodes % (8 * _NS) == 0
    mesh = plsc.VectorSubcoreMesh(core_axis_name="c", subcore_axis_name="s",
                                  num_cores=_NC, num_subcores=_NS)

    @functools.partial(
        pl.kernel,
        out_type=jax.ShapeDtypeStruct((_NC * n_nodes, hdim), jnp.float32),
        mesh=mesh,
        scratch_types=[
            pltpu.VMEM((nsteps, k), jnp.int32),
            pltpu.VMEM((nbuf, k, hdim), jnp.float32),
            pltpu.VMEM_SHARED((n_nodes, hdim), jnp.float32),
            pltpu.SemaphoreType.DMA,
            pltpu.SemaphoreType.DMA,
        ],
    )
    def scatter_kernel(msg_hbm, idx_hbm, zeros_hbm, out_hbm, idx_v, bufs, acc,
                       gsem, asem):
        c = lax.axis_index("c")
        s = lax.axis_index("s")
        w = c * _NS + s
        base = w * ec
        pltpu.sync_copy(zeros_hbm.at[pl.ds(s * rz, rz)], acc.at[pl.ds(s * rz, rz)])
        pltpu.sync_copy(idx_hbm.at[w], idx_v)
        plsc.subcore_barrier()
        for b in range(nbuf):
            pltpu.async_copy(
                msg_hbm.at[pl.ds(base + b * k, k)], bufs.at[b], gsem)

        def _step(step, b):
            pltpu.make_async_copy(
                msg_hbm.at[pl.ds(base, k)], bufs.at[b], gsem).wait()
            pltpu.async_copy(
                bufs.at[b], acc.at[idx_v.at[step]], asem, add=True)
            pltpu.make_async_copy(
                bufs.at[b], acc.at[idx_v.at[0]], asem).wait()
            nstep = step + nbuf

            @pl.when(nstep < nsteps)
            def _():
                pltpu.async_copy(
                    msg_hbm.at[pl.ds(base + nstep * k, k)], bufs.at[b], gsem)

        def group(gi, carry):
            for b in range(nbuf):
                _step(gi * nbuf + b, b)
            return carry

        lax.fori_loop(0, ngroups, group, 0)
        for t in range(ntail):
            _step(ngroups * nbuf + t, t)
        plsc.subcore_barrier()
        pltpu.sync_copy(acc.at[pl.ds(s * rz, rz)],
                        out_hbm.at[pl.ds(c * n_nodes + s * rz, rz)])

    return scatter_kernel


def kernel(x, edge_index, edge_attr, W_ne, b_ne, W_ee, b_ee, Wn, bn, Wg, bg,
           We, be, W1, b1, W2, b2):
    n, _ = x.shape
    e_cnt = edge_index.shape[1]
    hdim = W_ne.shape[1]
    n_layers = Wn.shape[0]
    gk = 80
    sk = 80
    n_pad = ((n + 639) // 640) * 640
    gather_fn = _make_sc_gather(2 * e_cnt, n, hdim, gk, nbuf=5)
    scatter_fn = _make_sc_scatter(e_cnt, n_pad, hdim, sk, nbuf=3)

    row = edge_index[0]
    col = edge_index[1]
    idx_all = jnp.concatenate([row, col]).reshape(
        _NW, (2 * e_cnt) // (_NW * gk), gk)
    col_r = col.reshape(_NW, e_cnt // (_NW * sk), sk)
    zeros = jnp.zeros((n_pad, hdim), jnp.float32)

    h = _linear(x, W_ne, b_ne)
    e = (edge_attr, W_ee, b_ee)

    for l in range(n_layers):
        wg_x, wg_e = Wg[l, :hdim], Wg[l, hdim:]
        wn_x, wn_e = Wn[l, :hdim], Wn[l, hdim:]
        we_r, we_c, we_e = We[l, :hdim], We[l, hdim:2 * hdim], We[l, 2 * hdim:]
        gath = gather_fn(h, idx_all)
        if l < n_layers - 1:
            wxi = jnp.concatenate([wg_x, we_c], axis=1)
            wxj = jnp.concatenate([wn_x, we_r], axis=1)
            wee = jnp.concatenate([wg_e, wn_e, we_e], axis=1)
            msg, e = _edge_layer(gath, e, wxi, wxj, wee, bg[l], bn[l], be[l],
                                 e_cnt, block_edges=2000)
            partials = scatter_fn(msg, col_r, zeros)
            h = _node_update(h, [partials], n_pad)
        else:
            wge = jnp.concatenate([wg_e, wn_e], axis=1)
            msg = _edge_layer_last(gath, e, wg_x, wn_x, wge, bg[l], bn[l],
                                   e_cnt, block_edges=2000)
            partials = scatter_fn(msg, col_r, zeros)
            out = _final(h, [partials], W1, b1, W2, b2, n_pad)
    return out


# speedup vs baseline: 1.0796x; 1.0796x over previous
import functools

import jax
import jax.numpy as jnp
from jax import lax
from jax.experimental import pallas as pl
from jax.experimental.pallas import tpu as pltpu
from jax.experimental.pallas import tpu_sc as plsc


def _linear_body(x_ref, w_ref, b_ref, o_ref):
    o_ref[...] = (
        jnp.dot(x_ref[...], w_ref[...], preferred_element_type=jnp.float32)
        + b_ref[...]
    )


def _linear(x, w, b, block_rows=None):
    m, k = x.shape
    _, n = w.shape
    if block_rows is None:
        block_rows = m
    grid = (m // block_rows,)
    return pl.pallas_call(
        _linear_body,
        grid=grid,
        in_specs=[
            pl.BlockSpec((block_rows, k), lambda i: (i, 0)),
            pl.BlockSpec((k, n), lambda i: (0, 0)),
            pl.BlockSpec((1, n), lambda i: (0, 0)),
        ],
        out_specs=pl.BlockSpec((block_rows, n), lambda i: (i, 0)),
        out_shape=jax.ShapeDtypeStruct((m, n), jnp.float32),
    )(x, w, b.reshape(1, -1))


def _edge_math(xj, xi, e, wxi_ref, wxj_ref, wee_ref, bg_ref, bn_ref, be_ref,
               msg_ref, eout_ref):
    h = e.shape[1]
    p_i = jnp.dot(xi, wxi_ref[...], preferred_element_type=jnp.float32)
    p_j = jnp.dot(xj, wxj_ref[...], preferred_element_type=jnp.float32)
    p_e = jnp.dot(e, wee_ref[...], preferred_element_type=jnp.float32)
    gate = jax.nn.sigmoid(p_i[:, :h] + p_e[:, :h] + bg_ref[...])
    msg_ref[...] = gate * (p_j[:, :h] + p_e[:, h:2 * h] + bn_ref[...])
    newe = p_i[:, h:] + p_j[:, h:] + p_e[:, 2 * h:] + be_ref[...]
    eout_ref[...] = (e + jnp.maximum(newe, 0.0)).astype(jnp.bfloat16)


def _edge_body(xj_ref, xi_ref, e_ref, wxi_ref, wxj_ref, wee_ref, bg_ref,
               bn_ref, be_ref, msg_ref, eout_ref):
    _edge_math(xj_ref[...], xi_ref[...],
               e_ref[...].astype(jnp.float32), wxi_ref, wxj_ref, wee_ref,
               bg_ref, bn_ref, be_ref, msg_ref, eout_ref)


def _edge_first_body(xj_ref, xi_ref, ea_ref, wea_ref, bea_ref, wxi_ref,
                     wxj_ref, wee_ref, bg_ref, bn_ref, be_ref, msg_ref,
                     eout_ref):
    e = (jnp.dot(ea_ref[...], wea_ref[...], preferred_element_type=jnp.float32)
         + bea_ref[...])
    _edge_math(xj_ref[...], xi_ref[...], e,
               wxi_ref, wxj_ref, wee_ref, bg_ref, bn_ref, be_ref, msg_ref,
               eout_ref)


def _edge_layer(gath, e, wxi, wxj, wee, bg, bn, be, n_edges, block_edges):
    hdim = wxi.shape[0]
    nblk = n_edges // block_edges
    first = isinstance(e, tuple)
    if first:
        e, wea, bea = e
        d_e = e.shape[1]
        extra_specs = [
            pl.BlockSpec((block_edges, d_e), lambda i: (i, 0)),
            pl.BlockSpec((d_e, hdim), lambda i: (0, 0)),
            pl.BlockSpec((1, hdim), lambda i: (0, 0)),
        ]
        extra_args = (e, wea, bea.reshape(1, -1))
        body = _edge_first_body
    else:
        extra_specs = [pl.BlockSpec((block_edges, hdim), lambda i: (i, 0))]
        extra_args = (e,)
        body = _edge_body
    return pl.pallas_call(
        body,
        grid=(nblk,),
        in_specs=[
            pl.BlockSpec((block_edges, hdim), lambda i: (i, 0)),
            pl.BlockSpec((block_edges, hdim), lambda i: (i + nblk, 0)),
            *extra_specs,
            pl.BlockSpec((hdim, 2 * hdim), lambda i: (0, 0)),
            pl.BlockSpec((hdim, 2 * hdim), lambda i: (0, 0)),
            pl.BlockSpec((hdim, 3 * hdim), lambda i: (0, 0)),
            pl.BlockSpec((1, hdim), lambda i: (0, 0)),
            pl.BlockSpec((1, hdim), lambda i: (0, 0)),
            pl.BlockSpec((1, hdim), lambda i: (0, 0)),
        ],
        out_specs=[
            pl.BlockSpec((block_edges, hdim), lambda i: (i, 0)),
            pl.BlockSpec((block_edges, hdim), lambda i: (i, 0)),
        ],
        out_shape=[
            jax.ShapeDtypeStruct((n_edges, hdim), jnp.float32),
            jax.ShapeDtypeStruct((n_edges, hdim), jnp.bfloat16),
        ],
    )(gath, gath, *extra_args, wxi, wxj, wee, bg.reshape(1, -1),
      bn.reshape(1, -1), be.reshape(1, -1))


def _edge_last_body(xj_ref, xi_ref, e_ref, wgx_ref, wnx_ref, wge_ref, bg_ref,
                    bn_ref, msg_ref):
    h = e_ref.shape[1]
    p_i = jnp.dot(xi_ref[...], wgx_ref[...], preferred_element_type=jnp.float32)
    p_j = jnp.dot(xj_ref[...], wnx_ref[...], preferred_element_type=jnp.float32)
    p_e = jnp.dot(e_ref[...].astype(jnp.float32), wge_ref[...],
                  preferred_element_type=jnp.float32)
    gate = jax.nn.sigmoid(p_i + p_e[:, :h] + bg_ref[...])
    msg_ref[...] = gate * (p_j + p_e[:, h:] + bn_ref[...])


def _edge_layer_last(gath, e, wgx, wnx, wge, bg, bn, n_edges, block_edges):
    hdim = e.shape[1]
    nblk = n_edges // block_edges
    return pl.pallas_call(
        _edge_last_body,
        grid=(nblk,),
        in_specs=[
            pl.BlockSpec((block_edges, hdim), lambda i: (i, 0)),
            pl.BlockSpec((block_edges, hdim), lambda i: (i + nblk, 0)),
            pl.BlockSpec((block_edges, hdim), lambda i: (i, 0)),
            pl.BlockSpec((hdim, hdim), lambda i: (0, 0)),
            pl.BlockSpec((hdim, hdim), lambda i: (0, 0)),
            pl.BlockSpec((hdim, 2 * hdim), lambda i: (0, 0)),
            pl.BlockSpec((1, hdim), lambda i: (0, 0)),
            pl.BlockSpec((1, hdim), lambda i: (0, 0)),
        ],
        out_specs=pl.BlockSpec((block_edges, hdim), lambda i: (i, 0)),
        out_shape=jax.ShapeDtypeStruct((n_edges, hdim), jnp.float32),
    )(gath, gath, e, wgx, wnx, wge, bg.reshape(1, -1), bn.reshape(1, -1))


def _node_update_body(h_ref, *refs):
    p_refs, (o_ref,) = refs[:-1], refs[-1:]
    acc = p_refs[0][...]
    for p in p_refs[1:]:
        acc = acc + p[...]
    o_ref[...] = h_ref[...] + jnp.maximum(acc, 0.0)


def _partial_specs(partials_list, n_pad, block_rows, hdim):
    p1_blk = n_pad // block_rows
    specs, args = [], []
    for p in partials_list:
        specs.append(pl.BlockSpec((block_rows, hdim), lambda i: (i, 0)))
        specs.append(
            pl.BlockSpec((block_rows, hdim), lambda i: (i + p1_blk, 0)))
        args.extend([p, p])
    return specs, args


def _node_update(h, partials_list, n_pad, block_rows=80):
    n, hdim = h.shape
    p_specs, p_args = _partial_specs(partials_list, n_pad, block_rows, hdim)
    return pl.pallas_call(
        _node_update_body,
        grid=(n // block_rows,),
        in_specs=[pl.BlockSpec((block_rows, hdim), lambda i: (i, 0)), *p_specs],
        out_specs=pl.BlockSpec((block_rows, hdim), lambda i: (i, 0)),
        out_shape=jax.ShapeDtypeStruct((n, hdim), jnp.float32),
    )(h, *p_args)


def _final_body(h_ref, *refs):
    *p_refs, w1_ref, b1_ref, w2_ref, b2_ref, o_ref = refs
    acc = p_refs[0][...]
    for p in p_refs[1:]:
        acc = acc + p[...]
    hn = h_ref[...] + jnp.maximum(acc, 0.0)
    t = jnp.maximum(
        jnp.dot(hn, w1_ref[...], preferred_element_type=jnp.float32)
        + b1_ref[...], 0.0)
    o_ref[...] = jnp.sum(t * w2_ref[...], axis=1, keepdims=True) + b2_ref[...]


def _final(h, partials_list, w1, b1, w2, b2, n_pad, block_rows=80):
    n, hdim = h.shape
    h2 = w1.shape[1]
    p_specs, p_args = _partial_specs(partials_list, n_pad, block_rows, hdim)
    return pl.pallas_call(
        _final_body,
        grid=(n // block_rows,),
        in_specs=[
            pl.BlockSpec((block_rows, hdim), lambda i: (i, 0)),
            *p_specs,
            pl.BlockSpec((hdim, h2), lambda i: (0, 0)),
            pl.BlockSpec((1, h2), lambda i: (0, 0)),
            pl.BlockSpec((1, h2), lambda i: (0, 0)),
            pl.BlockSpec((1, 1), lambda i: (0, 0)),
        ],
        out_specs=pl.BlockSpec((block_rows, 1), lambda i: (i, 0)),
        out_shape=jax.ShapeDtypeStruct((n, 1), jnp.float32),
    )(h, *p_args, w1, b1.reshape(1, -1), w2.reshape(1, -1),
      b2.reshape(1, 1))


_NC = 2
_NS = 16
_NW = _NC * _NS


def _make_sc_gather(n_idx, n_rows, hdim, k, nbuf, dtype=jnp.float32):
    ch = n_idx // _NW
    nsteps = ch // k
    assert ch % k == 0 and nsteps % nbuf == 0 and k % 16 == 0 and k <= 128
    mesh = plsc.VectorSubcoreMesh(core_axis_name="c", subcore_axis_name="s",
                                  num_cores=_NC, num_subcores=_NS)

    @functools.partial(
        pl.kernel,
        out_type=jax.ShapeDtypeStruct((n_idx, hdim), dtype),
        mesh=mesh,
        scratch_types=[
            pltpu.VMEM((nsteps, k), jnp.int32),
            pltpu.VMEM((nbuf, k, hdim), dtype),
            pltpu.SemaphoreType.DMA,
            pltpu.SemaphoreType.DMA,
        ],
    )
    def gather_kernel(table_hbm, idx_hbm, out_hbm, idx_v, bufs, gsem, ssem):
        c = lax.axis_index("c")
        s = lax.axis_index("s")
        w = c * _NS + s
        base = w * ch
        pltpu.sync_copy(idx_hbm.at[w], idx_v)
        for b in range(nbuf):
            pltpu.async_copy(table_hbm.at[idx_v.at[b]], bufs.at[b], gsem)

        def group(gi, carry):
            for b in range(nbuf):
                step = gi * nbuf + b
                pltpu.make_async_copy(
                    table_hbm.at[idx_v.at[0]], bufs.at[b], gsem).wait()
                pltpu.async_copy(
                    bufs.at[b], out_hbm.at[pl.ds(base + step * k, k)], ssem)
                pltpu.make_async_copy(
                    bufs.at[b], out_hbm.at[pl.ds(base, k)], ssem).wait()
                nstep = step + nbuf

                @pl.when(nstep < nsteps)
                def _():
                    pltpu.async_copy(
                        table_hbm.at[idx_v.at[nstep]], bufs.at[b], gsem)
            return carry

        lax.fori_loop(0, nsteps // nbuf, group, 0)

    return gather_kernel


def _make_sc_scatter(n_edges, n_nodes, hdim, k, nbuf):
    ec = n_edges // _NW
    nsteps = ec // k
    rz = n_nodes // _NS
    ngroups, ntail = divmod(nsteps, nbuf)
    assert ec % k == 0 and n_nodes % (8 * _NS) == 0
    mesh = plsc.VectorSubcoreMesh(core_axis_name="c", subcore_axis_name="s",
                                  num_cores=_NC, num_subcores=_NS)

    @functools.partial(
        pl.kernel,
        out_type=jax.ShapeDtypeStruct((_NC * n_nodes, hdim), jnp.float32),
        mesh=mesh,
        scratch_types=[
            pltpu.VMEM((nsteps, k), jnp.int32),
            pltpu.VMEM((nbuf, k, hdim), jnp.float32),
            pltpu.VMEM_SHARED((n_nodes, hdim), jnp.float32),
            pltpu.SemaphoreType.DMA,
            pltpu.SemaphoreType.DMA,
        ],
    )
    def scatter_kernel(msg_hbm, idx_hbm, zeros_hbm, out_hbm, idx_v, bufs, acc,
                       gsem, asem):
        c = lax.axis_index("c")
        s = lax.axis_index("s")
        w = c * _NS + s
        base = w * ec
        pltpu.sync_copy(zeros_hbm.at[pl.ds(s * rz, rz)], acc.at[pl.ds(s * rz, rz)])
        pltpu.sync_copy(idx_hbm.at[w], idx_v)
        plsc.subcore_barrier()
        for b in range(nbuf):
            pltpu.async_copy(
                msg_hbm.at[pl.ds(base + b * k, k)], bufs.at[b], gsem)

        def _step(step, b):
            pltpu.make_async_copy(
                msg_hbm.at[pl.ds(base, k)], bufs.at[b], gsem).wait()
            pltpu.async_copy(
                bufs.at[b], acc.at[idx_v.at[step]], asem, add=True)
            pltpu.make_async_copy(
                bufs.at[b], acc.at[idx_v.at[0]], asem).wait()
            nstep = step + nbuf

            @pl.when(nstep < nsteps)
            def _():
                pltpu.async_copy(
                    msg_hbm.at[pl.ds(base + nstep * k, k)], bufs.at[b], gsem)

        def group(gi, carry):
            for b in range(nbuf):
                _step(gi * nbuf + b, b)
            return carry

        lax.fori_loop(0, ngroups, group, 0)
        for t in range(ntail):
            _step(ngroups * nbuf + t, t)
        plsc.subcore_barrier()
        pltpu.sync_copy(acc.at[pl.ds(s * rz, rz)],
                        out_hbm.at[pl.ds(c * n_nodes + s * rz, rz)])

    return scatter_kernel


def kernel(x, edge_index, edge_attr, W_ne, b_ne, W_ee, b_ee, Wn, bn, Wg, bg,
           We, be, W1, b1, W2, b2):
    n, _ = x.shape
    e_cnt = edge_index.shape[1]
    hdim = W_ne.shape[1]
    n_layers = Wn.shape[0]
    eh = e_cnt // 2

    gk = 80
    sk = 40
    beh = 4000
    n_pad = ((n + 639) // 640) * 640
    gather_fn = _make_sc_gather(2 * eh, n, hdim, gk, nbuf=5)
    scatter_fn = _make_sc_scatter(eh, n_pad, hdim, sk, nbuf=3)

    row = edge_index[0]
    col = edge_index[1]
    halves = []
    for half in range(2):
        sl = slice(half * eh, (half + 1) * eh)
        idx_h = jnp.concatenate([row[sl], col[sl]]).reshape(
            _NW, (2 * eh) // (_NW * gk), gk)
        col_h = col[sl].reshape(_NW, eh // (_NW * sk), sk)
        halves.append((idx_h, col_h, edge_attr[sl]))
    zeros = jnp.zeros((n_pad, hdim), jnp.float32)

    h = _linear(x, W_ne, b_ne)
    e_half = [(ea, W_ee, b_ee) for (_, _, ea) in halves]

    for l in range(n_layers):
        wg_x, wg_e = Wg[l, :hdim], Wg[l, hdim:]
        wn_x, wn_e = Wn[l, :hdim], Wn[l, hdim:]
        we_r, we_c, we_e = We[l, :hdim], We[l, hdim:2 * hdim], We[l, 2 * hdim:]
        gaths = [gather_fn(h, halves[0][0]), gather_fn(h, halves[1][0])]
        partials = []
        if l < n_layers - 1:
            wxi = jnp.concatenate([wg_x, we_c], axis=1)
            wxj = jnp.concatenate([wn_x, we_r], axis=1)
            wee = jnp.concatenate([wg_e, wn_e, we_e], axis=1)
            for half in range(2):
                msg, e_half[half] = _edge_layer(
                    gaths[half], e_half[half], wxi, wxj, wee, bg[l], bn[l],
                    be[l], eh, block_edges=beh)
                partials.append(scatter_fn(msg, halves[half][1], zeros))
            h = _node_update(h, partials, n_pad)
        else:
            wge = jnp.concatenate([wg_e, wn_e], axis=1)
            for half in range(2):
                msg = _edge_layer_last(
                    gaths[half], e_half[half], wg_x, wn_x, wge, bg[l], bn[l],
                    eh, block_edges=beh)
                partials.append(scatter_fn(msg, halves[half][1], zeros))
            out = _final(h, partials, W1, b1, W2, b2, n_pad)
    return out


# speedup vs baseline: 1.0857x; 1.0057x over previous
import functools

import jax
import jax.numpy as jnp
from jax import lax
from jax.experimental import pallas as pl
from jax.experimental.pallas import tpu as pltpu
from jax.experimental.pallas import tpu_sc as plsc


def _linear_body(x_ref, w_ref, b_ref, o_ref):
    o_ref[...] = (
        jnp.dot(x_ref[...], w_ref[...], preferred_element_type=jnp.float32)
        + b_ref[...]
    )


def _linear(x, w, b, block_rows=None):
    m, k = x.shape
    _, n = w.shape
    if block_rows is None:
        block_rows = m
    grid = (m // block_rows,)
    return pl.pallas_call(
        _linear_body,
        grid=grid,
        in_specs=[
            pl.BlockSpec((block_rows, k), lambda i: (i, 0)),
            pl.BlockSpec((k, n), lambda i: (0, 0)),
            pl.BlockSpec((1, n), lambda i: (0, 0)),
        ],
        out_specs=pl.BlockSpec((block_rows, n), lambda i: (i, 0)),
        out_shape=jax.ShapeDtypeStruct((m, n), jnp.float32),
    )(x, w, b.reshape(1, -1))


def _edge_math(xj, xi, e, wxi_ref, wxj_ref, wee_ref, bg_ref, bn_ref, be_ref,
               msg_ref, eout_ref):
    h = e.shape[1]
    p_i = jnp.dot(xi, wxi_ref[...], preferred_element_type=jnp.float32)
    p_j = jnp.dot(xj, wxj_ref[...], preferred_element_type=jnp.float32)
    p_e = jnp.dot(e, wee_ref[...], preferred_element_type=jnp.float32)
    gate = jax.nn.sigmoid(p_i[:, :h] + p_e[:, :h] + bg_ref[...])
    msg_ref[...] = gate * (p_j[:, :h] + p_e[:, h:2 * h] + bn_ref[...])
    newe = p_i[:, h:] + p_j[:, h:] + p_e[:, 2 * h:] + be_ref[...]
    eout_ref[...] = (e + jnp.maximum(newe, 0.0)).astype(jnp.bfloat16)


def _edge_body(xj_ref, xi_ref, e_ref, wxi_ref, wxj_ref, wee_ref, bg_ref,
               bn_ref, be_ref, msg_ref, eout_ref):
    _edge_math(xj_ref[...], xi_ref[...],
               e_ref[...].astype(jnp.float32), wxi_ref, wxj_ref, wee_ref,
               bg_ref, bn_ref, be_ref, msg_ref, eout_ref)


def _edge_first_body(xj_ref, xi_ref, ea_ref, wea_ref, bea_ref, wxi_ref,
                     wxj_ref, wee_ref, bg_ref, bn_ref, be_ref, msg_ref,
                     eout_ref):
    e = (jnp.dot(ea_ref[...], wea_ref[...], preferred_element_type=jnp.float32)
         + bea_ref[...])
    _edge_math(xj_ref[...], xi_ref[...], e,
               wxi_ref, wxj_ref, wee_ref, bg_ref, bn_ref, be_ref, msg_ref,
               eout_ref)


def _edge_layer(gath, e, wxi, wxj, wee, bg, bn, be, n_edges, block_edges):
    hdim = wxi.shape[0]
    nblk = n_edges // block_edges
    first = isinstance(e, tuple)
    if first:
        e, wea, bea = e
        d_e = e.shape[1]
        extra_specs = [
            pl.BlockSpec((block_edges, d_e), lambda i: (i, 0)),
            pl.BlockSpec((d_e, hdim), lambda i: (0, 0)),
            pl.BlockSpec((1, hdim), lambda i: (0, 0)),
        ]
        extra_args = (e, wea, bea.reshape(1, -1))
        body = _edge_first_body
    else:
        extra_specs = [pl.BlockSpec((block_edges, hdim), lambda i: (i, 0))]
        extra_args = (e,)
        body = _edge_body
    return pl.pallas_call(
        body,
        grid=(nblk,),
        in_specs=[
            pl.BlockSpec((block_edges, hdim), lambda i: (i, 0)),
            pl.BlockSpec((block_edges, hdim), lambda i: (i + nblk, 0)),
            *extra_specs,
            pl.BlockSpec((hdim, 2 * hdim), lambda i: (0, 0)),
            pl.BlockSpec((hdim, 2 * hdim), lambda i: (0, 0)),
            pl.BlockSpec((hdim, 3 * hdim), lambda i: (0, 0)),
            pl.BlockSpec((1, hdim), lambda i: (0, 0)),
            pl.BlockSpec((1, hdim), lambda i: (0, 0)),
            pl.BlockSpec((1, hdim), lambda i: (0, 0)),
        ],
        out_specs=[
            pl.BlockSpec((block_edges, hdim), lambda i: (i, 0)),
            pl.BlockSpec((block_edges, hdim), lambda i: (i, 0)),
        ],
        out_shape=[
            jax.ShapeDtypeStruct((n_edges, hdim), jnp.float32),
            jax.ShapeDtypeStruct((n_edges, hdim), jnp.bfloat16),
        ],
    )(gath, gath, *extra_args, wxi, wxj, wee, bg.reshape(1, -1),
      bn.reshape(1, -1), be.reshape(1, -1))


def _edge_last_body(xj_ref, xi_ref, e_ref, wgx_ref, wnx_ref, wge_ref, bg_ref,
                    bn_ref, msg_ref):
    h = e_ref.shape[1]
    p_i = jnp.dot(xi_ref[...], wgx_ref[...], preferred_element_type=jnp.float32)
    p_j = jnp.dot(xj_ref[...], wnx_ref[...], preferred_element_type=jnp.float32)
    p_e = jnp.dot(e_ref[...].astype(jnp.float32), wge_ref[...],
                  preferred_element_type=jnp.float32)
    gate = jax.nn.sigmoid(p_i + p_e[:, :h] + bg_ref[...])
    msg_ref[...] = gate * (p_j + p_e[:, h:] + bn_ref[...])


def _edge_layer_last(gath, e, wgx, wnx, wge, bg, bn, n_edges, block_edges):
    hdim = e.shape[1]
    nblk = n_edges // block_edges
    return pl.pallas_call(
        _edge_last_body,
        grid=(nblk,),
        in_specs=[
            pl.BlockSpec((block_edges, hdim), lambda i: (i, 0)),
            pl.BlockSpec((block_edges, hdim), lambda i: (i + nblk, 0)),
            pl.BlockSpec((block_edges, hdim), lambda i: (i, 0)),
            pl.BlockSpec((hdim, hdim), lambda i: (0, 0)),
            pl.BlockSpec((hdim, hdim), lambda i: (0, 0)),
            pl.BlockSpec((hdim, 2 * hdim), lambda i: (0, 0)),
            pl.BlockSpec((1, hdim), lambda i: (0, 0)),
            pl.BlockSpec((1, hdim), lambda i: (0, 0)),
        ],
        out_specs=pl.BlockSpec((block_edges, hdim), lambda i: (i, 0)),
        out_shape=jax.ShapeDtypeStruct((n_edges, hdim), jnp.float32),
    )(gath, gath, e, wgx, wnx, wge, bg.reshape(1, -1), bn.reshape(1, -1))


def _node_update_body(h_ref, *refs):
    p_refs, (o_ref,) = refs[:-1], refs[-1:]
    acc = p_refs[0][...]
    for p in p_refs[1:]:
        acc = acc + p[...]
    o_ref[...] = h_ref[...] + jnp.maximum(acc, 0.0)


def _partial_specs(partials_list, n_pad, block_rows, hdim):
    p1_blk = n_pad // block_rows
    specs, args = [], []
    for p in partials_list:
        specs.append(pl.BlockSpec((block_rows, hdim), lambda i: (i, 0)))
        specs.append(
            pl.BlockSpec((block_rows, hdim), lambda i: (i + p1_blk, 0)))
        args.extend([p, p])
    return specs, args


def _node_update(h, partials_list, n_pad, block_rows=80):
    n, hdim = h.shape
    p_specs, p_args = _partial_specs(partials_list, n_pad, block_rows, hdim)
    return pl.pallas_call(
        _node_update_body,
        grid=(n // block_rows,),
        in_specs=[pl.BlockSpec((block_rows, hdim), lambda i: (i, 0)), *p_specs],
        out_specs=pl.BlockSpec((block_rows, hdim), lambda i: (i, 0)),
        out_shape=jax.ShapeDtypeStruct((n, hdim), jnp.float32),
    )(h, *p_args)


def _final_body(h_ref, *refs):
    *p_refs, w1_ref, b1_ref, w2_ref, b2_ref, o_ref = refs
    acc = p_refs[0][...]
    for p in p_refs[1:]:
        acc = acc + p[...]
    hn = h_ref[...] + jnp.maximum(acc, 0.0)
    t = jnp.maximum(
        jnp.dot(hn, w1_ref[...], preferred_element_type=jnp.float32)
        + b1_ref[...], 0.0)
    o_ref[...] = jnp.sum(t * w2_ref[...], axis=1, keepdims=True) + b2_ref[...]


def _final(h, partials_list, w1, b1, w2, b2, n_pad, block_rows=80):
    n, hdim = h.shape
    h2 = w1.shape[1]
    p_specs, p_args = _partial_specs(partials_list, n_pad, block_rows, hdim)
    return pl.pallas_call(
        _final_body,
        grid=(n // block_rows,),
        in_specs=[
            pl.BlockSpec((block_rows, hdim), lambda i: (i, 0)),
            *p_specs,
            pl.BlockSpec((hdim, h2), lambda i: (0, 0)),
            pl.BlockSpec((1, h2), lambda i: (0, 0)),
            pl.BlockSpec((1, h2), lambda i: (0, 0)),
            pl.BlockSpec((1, 1), lambda i: (0, 0)),
        ],
        out_specs=pl.BlockSpec((block_rows, 1), lambda i: (i, 0)),
        out_shape=jax.ShapeDtypeStruct((n, 1), jnp.float32),
    )(h, *p_args, w1, b1.reshape(1, -1), w2.reshape(1, -1),
      b2.reshape(1, 1))


_NC = 2
_NS = 16
_NW = _NC * _NS


def _make_sc_gather(n_idx, n_rows, hdim, k, nbuf, dtype=jnp.float32):
    ch = n_idx // _NW
    nsteps = ch // k
    assert ch % k == 0 and nsteps % nbuf == 0 and k % 16 == 0 and k <= 128
    mesh = plsc.VectorSubcoreMesh(core_axis_name="c", subcore_axis_name="s",
                                  num_cores=_NC, num_subcores=_NS)

    @functools.partial(
        pl.kernel,
        out_type=jax.ShapeDtypeStruct((n_idx, hdim), dtype),
        mesh=mesh,
        scratch_types=[
            pltpu.VMEM((nsteps, k), jnp.int32),
            pltpu.VMEM((nbuf, k, hdim), dtype),
            pltpu.SemaphoreType.DMA,
            pltpu.SemaphoreType.DMA,
        ],
    )
    def gather_kernel(table_hbm, idx_hbm, out_hbm, idx_v, bufs, gsem, ssem):
        c = lax.axis_index("c")
        s = lax.axis_index("s")
        w = c * _NS + s
        base = w * ch
        pltpu.sync_copy(idx_hbm.at[w], idx_v)
        for b in range(nbuf):
            pltpu.async_copy(table_hbm.at[idx_v.at[b]], bufs.at[b], gsem)

        def group(gi, carry):
            for b in range(nbuf):
                step = gi * nbuf + b
                pltpu.make_async_copy(
                    table_hbm.at[idx_v.at[0]], bufs.at[b], gsem).wait()
                pltpu.async_copy(
                    bufs.at[b], out_hbm.at[pl.ds(base + step * k, k)], ssem)
                pltpu.make_async_copy(
                    bufs.at[b], out_hbm.at[pl.ds(base, k)], ssem).wait()
                nstep = step + nbuf

                @pl.when(nstep < nsteps)
                def _():
                    pltpu.async_copy(
                        table_hbm.at[idx_v.at[nstep]], bufs.at[b], gsem)
            return carry

        lax.fori_loop(0, nsteps // nbuf, group, 0)

    return gather_kernel


def _make_sc_scatter(n_edges, n_nodes, hdim, k, nbuf):
    ec = n_edges // _NW
    nsteps = ec // k
    rz = n_nodes // _NS
    ngroups, ntail = divmod(nsteps, nbuf)
    assert ec % k == 0 and n_nodes % (8 * _NS) == 0
    mesh = plsc.VectorSubcoreMesh(core_axis_name="c", subcore_axis_name="s",
                                  num_cores=_NC, num_subcores=_NS)

    @functools.partial(
        pl.kernel,
        out_type=jax.ShapeDtypeStruct((_NC * n_nodes, hdim), jnp.float32),
        mesh=mesh,
        scratch_types=[
            pltpu.VMEM((nsteps, k), jnp.int32),
            pltpu.VMEM((nbuf, k, hdim), jnp.float32),
            pltpu.VMEM_SHARED((n_nodes, hdim), jnp.float32),
            pltpu.SemaphoreType.DMA,
            pltpu.SemaphoreType.DMA,
        ],
    )
    def scatter_kernel(msg_hbm, idx_hbm, zeros_hbm, out_hbm, idx_v, bufs, acc,
                       gsem, asem):
        c = lax.axis_index("c")
        s = lax.axis_index("s")
        w = c * _NS + s
        base = w * ec
        pltpu.sync_copy(zeros_hbm.at[pl.ds(s * rz, rz)], acc.at[pl.ds(s * rz, rz)])
        pltpu.sync_copy(idx_hbm.at[w], idx_v)
        plsc.subcore_barrier()
        for b in range(nbuf):
            pltpu.async_copy(
                msg_hbm.at[pl.ds(base + b * k, k)], bufs.at[b], gsem)

        def _step(step, b):
            pltpu.make_async_copy(
                msg_hbm.at[pl.ds(base, k)], bufs.at[b], gsem).wait()
            pltpu.async_copy(
                bufs.at[b], acc.at[idx_v.at[step]], asem, add=True)
            pltpu.make_async_copy(
                bufs.at[b], acc.at[idx_v.at[0]], asem).wait()
            nstep = step + nbuf

            @pl.when(nstep < nsteps)
            def _():
                pltpu.async_copy(
                    msg_hbm.at[pl.ds(base + nstep * k, k)], bufs.at[b], gsem)

        def group(gi, carry):
            for b in range(nbuf):
                _step(gi * nbuf + b, b)
            return carry

        lax.fori_loop(0, ngroups, group, 0)
        for t in range(ntail):
            _step(ngroups * nbuf + t, t)
        plsc.subcore_barrier()
        pltpu.sync_copy(acc.at[pl.ds(s * rz, rz)],
                        out_hbm.at[pl.ds(c * n_nodes + s * rz, rz)])

    return scatter_kernel


def kernel(x, edge_index, edge_attr, W_ne, b_ne, W_ee, b_ee, Wn, bn, Wg, bg,
           We, be, W1, b1, W2, b2):
    n, _ = x.shape
    e_cnt = edge_index.shape[1]
    hdim = W_ne.shape[1]
    n_layers = Wn.shape[0]
    eh = e_cnt // 2

    gk = 80
    sk = 40
    beh = 8000
    n_pad = ((n + 639) // 640) * 640
    gather_fn = _make_sc_gather(2 * eh, n, hdim, gk, nbuf=5)
    scatter_fn = _make_sc_scatter(eh, n_pad, hdim, sk, nbuf=3)

    row = edge_index[0]
    col = edge_index[1]
    halves = []
    for half in range(2):
        sl = slice(half * eh, (half + 1) * eh)
        idx_h = jnp.concatenate([row[sl], col[sl]]).reshape(
            _NW, (2 * eh) // (_NW * gk), gk)
        col_h = col[sl].reshape(_NW, eh // (_NW * sk), sk)
        halves.append((idx_h, col_h, edge_attr[sl]))
    zeros = jnp.zeros((n_pad, hdim), jnp.float32)

    h = _linear(x, W_ne, b_ne)
    e_half = [(ea, W_ee, b_ee) for (_, _, ea) in halves]

    for l in range(n_layers):
        wg_x, wg_e = Wg[l, :hdim], Wg[l, hdim:]
        wn_x, wn_e = Wn[l, :hdim], Wn[l, hdim:]
        we_r, we_c, we_e = We[l, :hdim], We[l, hdim:2 * hdim], We[l, 2 * hdim:]
        gaths = [gather_fn(h, halves[0][0]), gather_fn(h, halves[1][0])]
        partials = []
        if l < n_layers - 1:
            wxi = jnp.concatenate([wg_x, we_c], axis=1)
            wxj = jnp.concatenate([wn_x, we_r], axis=1)
            wee = jnp.concatenate([wg_e, wn_e, we_e], axis=1)
            for half in range(2):
                msg, e_half[half] = _edge_layer(
                    gaths[half], e_half[half], wxi, wxj, wee, bg[l], bn[l],
                    be[l], eh, block_edges=beh)
                partials.append(scatter_fn(msg, halves[half][1], zeros))
            h = _node_update(h, partials, n_pad)
        else:
            wge = jnp.concatenate([wg_e, wn_e], axis=1)
            for half in range(2):
                msg = _edge_layer_last(
                    gaths[half], e_half[half], wg_x, wn_x, wge, bg[l], bn[l],
                    eh, block_edges=beh)
                partials.append(scatter_fn(msg, halves[half][1], zeros))
            out = _final(h, partials, W1, b1, W2, b2, n_pad)
    return out


# speedup vs baseline: 1.3164x; 1.2125x over previous
import functools

import jax
import jax.numpy as jnp
from jax import lax
from jax.experimental import pallas as pl
from jax.experimental.pallas import tpu as pltpu
from jax.experimental.pallas import tpu_sc as plsc


def _linear_body(x_ref, w_ref, b_ref, o_ref):
    o_ref[...] = (
        jnp.dot(x_ref[...], w_ref[...], preferred_element_type=jnp.float32)
        + b_ref[...]
    )


def _linear(x, w, b, block_rows=None):
    m, k = x.shape
    _, n = w.shape
    if block_rows is None:
        block_rows = m
    grid = (m // block_rows,)
    return pl.pallas_call(
        _linear_body,
        grid=grid,
        in_specs=[
            pl.BlockSpec((block_rows, k), lambda i: (i, 0)),
            pl.BlockSpec((k, n), lambda i: (0, 0)),
            pl.BlockSpec((1, n), lambda i: (0, 0)),
        ],
        out_specs=pl.BlockSpec((block_rows, n), lambda i: (i, 0)),
        out_shape=jax.ShapeDtypeStruct((m, n), jnp.float32),
    )(x, w, b.reshape(1, -1))


def _edge_math(xj, xi, e, wxi_ref, wxj_ref, wee_ref, bg_ref, bn_ref, be_ref,
               msg_ref, eout_ref):
    h = e.shape[1]
    p_i = jnp.dot(xi, wxi_ref[...], preferred_element_type=jnp.float32)
    p_j = jnp.dot(xj, wxj_ref[...], preferred_element_type=jnp.float32)
    p_e = jnp.dot(e, wee_ref[...], preferred_element_type=jnp.float32)
    gate = jax.nn.sigmoid(p_i[:, :h] + p_e[:, :h] + bg_ref[...])
    msg_ref[...] = gate * (p_j[:, :h] + p_e[:, h:2 * h] + bn_ref[...])
    newe = p_i[:, h:] + p_j[:, h:] + p_e[:, 2 * h:] + be_ref[...]
    eout_ref[...] = (e + jnp.maximum(newe, 0.0)).astype(jnp.bfloat16)


def _edge_body(xj_ref, xi_ref, e_ref, wxi_ref, wxj_ref, wee_ref, bg_ref,
               bn_ref, be_ref, msg_ref, eout_ref):
    _edge_math(xj_ref[...], xi_ref[...],
               e_ref[...].astype(jnp.float32), wxi_ref, wxj_ref, wee_ref,
               bg_ref, bn_ref, be_ref, msg_ref, eout_ref)


def _edge_first_body(xj_ref, xi_ref, ea_ref, wea_ref, bea_ref, wxi_ref,
                     wxj_ref, wee_ref, bg_ref, bn_ref, be_ref, msg_ref,
                     eout_ref):
    e = (jnp.dot(ea_ref[...], wea_ref[...], preferred_element_type=jnp.float32)
         + bea_ref[...])
    _edge_math(xj_ref[...], xi_ref[...], e,
               wxi_ref, wxj_ref, wee_ref, bg_ref, bn_ref, be_ref, msg_ref,
               eout_ref)


def _edge_layer(gath, e, wxi, wxj, wee, bg, bn, be, n_edges, block_edges):
    hdim = wxi.shape[0]
    nblk = n_edges // block_edges
    first = isinstance(e, tuple)
    if first:
        e, wea, bea = e
        d_e = e.shape[1]
        extra_specs = [
            pl.BlockSpec((block_edges, d_e), lambda i: (i, 0)),
            pl.BlockSpec((d_e, hdim), lambda i: (0, 0)),
            pl.BlockSpec((1, hdim), lambda i: (0, 0)),
        ]
        extra_args = (e, wea, bea.reshape(1, -1))
        body = _edge_first_body
    else:
        extra_specs = [pl.BlockSpec((block_edges, hdim), lambda i: (i, 0))]
        extra_args = (e,)
        body = _edge_body
    return pl.pallas_call(
        body,
        grid=(nblk,),
        in_specs=[
            pl.BlockSpec((block_edges, hdim), lambda i: (i, 0)),
            pl.BlockSpec((block_edges, hdim), lambda i: (i + nblk, 0)),
            *extra_specs,
            pl.BlockSpec((hdim, 2 * hdim), lambda i: (0, 0)),
            pl.BlockSpec((hdim, 2 * hdim), lambda i: (0, 0)),
            pl.BlockSpec((hdim, 3 * hdim), lambda i: (0, 0)),
            pl.BlockSpec((1, hdim), lambda i: (0, 0)),
            pl.BlockSpec((1, hdim), lambda i: (0, 0)),
            pl.BlockSpec((1, hdim), lambda i: (0, 0)),
        ],
        out_specs=[
            pl.BlockSpec((block_edges, hdim), lambda i: (i, 0)),
            pl.BlockSpec((block_edges, hdim), lambda i: (i, 0)),
        ],
        out_shape=[
            jax.ShapeDtypeStruct((n_edges, hdim), jnp.float32),
            jax.ShapeDtypeStruct((n_edges, hdim), jnp.bfloat16),
        ],
    )(gath, gath, *extra_args, wxi, wxj, wee, bg.reshape(1, -1),
      bn.reshape(1, -1), be.reshape(1, -1))


def _edge_last_body(xj_ref, xi_ref, e_ref, wgx_ref, wnx_ref, wge_ref, bg_ref,
                    bn_ref, msg_ref):
    h = e_ref.shape[1]
    p_i = jnp.dot(xi_ref[...], wgx_ref[...], preferred_element_type=jnp.float32)
    p_j = jnp.dot(xj_ref[...], wnx_ref[...], preferred_element_type=jnp.float32)
    p_e = jnp.dot(e_ref[...].astype(jnp.float32), wge_ref[...],
                  preferred_element_type=jnp.float32)
    gate = jax.nn.sigmoid(p_i + p_e[:, :h] + bg_ref[...])
    msg_ref[...] = gate * (p_j + p_e[:, h:] + bn_ref[...])


def _edge_layer_last(gath, e, wgx, wnx, wge, bg, bn, n_edges, block_edges):
    hdim = e.shape[1]
    nblk = n_edges // block_edges
    return pl.pallas_call(
        _edge_last_body,
        grid=(nblk,),
        in_specs=[
            pl.BlockSpec((block_edges, hdim), lambda i: (i, 0)),
            pl.BlockSpec((block_edges, hdim), lambda i: (i + nblk, 0)),
            pl.BlockSpec((block_edges, hdim), lambda i: (i, 0)),
            pl.BlockSpec((hdim, hdim), lambda i: (0, 0)),
            pl.BlockSpec((hdim, hdim), lambda i: (0, 0)),
            pl.BlockSpec((hdim, 2 * hdim), lambda i: (0, 0)),
            pl.BlockSpec((1, hdim), lambda i: (0, 0)),
            pl.BlockSpec((1, hdim), lambda i: (0, 0)),
        ],
        out_specs=pl.BlockSpec((block_edges, hdim), lambda i: (i, 0)),
        out_shape=jax.ShapeDtypeStruct((n_edges, hdim), jnp.float32),
    )(gath, gath, e, wgx, wnx, wge, bg.reshape(1, -1), bn.reshape(1, -1))


def _node_update_body(h_ref, *refs):
    p_refs, (o_ref,) = refs[:-1], refs[-1:]
    acc = p_refs[0][...]
    for p in p_refs[1:]:
        acc = acc + p[...]
    o_ref[...] = h_ref[...] + jnp.maximum(acc, 0.0)


def _partial_specs(partials_list, n_pad, block_rows, hdim):
    p1_blk = n_pad // block_rows
    specs, args = [], []
    for p in partials_list:
        specs.append(pl.BlockSpec((block_rows, hdim), lambda i: (i, 0)))
        specs.append(
            pl.BlockSpec((block_rows, hdim), lambda i: (i + p1_blk, 0)))
        args.extend([p, p])
    return specs, args


def _node_update(h, partials_list, n_pad, block_rows=80):
    n, hdim = h.shape
    p_specs, p_args = _partial_specs(partials_list, n_pad, block_rows, hdim)
    return pl.pallas_call(
        _node_update_body,
        grid=(n // block_rows,),
        in_specs=[pl.BlockSpec((block_rows, hdim), lambda i: (i, 0)), *p_specs],
        out_specs=pl.BlockSpec((block_rows, hdim), lambda i: (i, 0)),
        out_shape=jax.ShapeDtypeStruct((n, hdim), jnp.float32),
    )(h, *p_args)


def _final_body(h_ref, *refs):
    *p_refs, w1_ref, b1_ref, w2_ref, b2_ref, o_ref = refs
    acc = p_refs[0][...]
    for p in p_refs[1:]:
        acc = acc + p[...]
    hn = h_ref[...] + jnp.maximum(acc, 0.0)
    t = jnp.maximum(
        jnp.dot(hn, w1_ref[...], preferred_element_type=jnp.float32)
        + b1_ref[...], 0.0)
    o_ref[...] = jnp.sum(t * w2_ref[...], axis=1, keepdims=True) + b2_ref[...]


def _final(h, partials_list, w1, b1, w2, b2, n_pad, block_rows=80):
    n, hdim = h.shape
    h2 = w1.shape[1]
    p_specs, p_args = _partial_specs(partials_list, n_pad, block_rows, hdim)
    return pl.pallas_call(
        _final_body,
        grid=(n // block_rows,),
        in_specs=[
            pl.BlockSpec((block_rows, hdim), lambda i: (i, 0)),
            *p_specs,
            pl.BlockSpec((hdim, h2), lambda i: (0, 0)),
            pl.BlockSpec((1, h2), lambda i: (0, 0)),
            pl.BlockSpec((1, h2), lambda i: (0, 0)),
            pl.BlockSpec((1, 1), lambda i: (0, 0)),
        ],
        out_specs=pl.BlockSpec((block_rows, 1), lambda i: (i, 0)),
        out_shape=jax.ShapeDtypeStruct((n, 1), jnp.float32),
    )(h, *p_args, w1, b1.reshape(1, -1), w2.reshape(1, -1),
      b2.reshape(1, 1))


_NC = 2
_NS = 16
_NW = _NC * _NS


def _make_sc_gather(n_idx, n_rows, hdim, k, nbuf, dtype=jnp.float32,
                    spmem_table=False):
    ch = n_idx // _NW
    nsteps = ch // k
    ngroups, ntail = divmod(nsteps, nbuf)
    assert ch % k == 0 and k % 16 == 0 and k <= 128
    mesh = plsc.VectorSubcoreMesh(core_axis_name="c", subcore_axis_name="s",
                                  num_cores=_NC, num_subcores=_NS)
    rstage = (n_rows // _NS) // 8 * 8
    rtail = n_rows - rstage * _NS
    scratch = [
        pltpu.VMEM((nsteps, k), jnp.int32),
        pltpu.VMEM((nbuf, k, hdim), dtype),
        pltpu.SemaphoreType.DMA,
        pltpu.SemaphoreType.DMA,
    ]
    if spmem_table:
        scratch.insert(2, pltpu.VMEM_SHARED((n_rows, hdim), dtype))

    @functools.partial(
        pl.kernel,
        out_type=jax.ShapeDtypeStruct((n_idx, hdim), dtype),
        mesh=mesh,
        scratch_types=scratch,
    )
    def gather_kernel(table_hbm, idx_hbm, out_hbm, idx_v, bufs, *rest):
        if spmem_table:
            tbl, gsem, ssem = rest
        else:
            gsem, ssem = rest
            tbl = table_hbm
        c = lax.axis_index("c")
        s = lax.axis_index("s")
        w = c * _NS + s
        base = w * ch
        pltpu.sync_copy(idx_hbm.at[w], idx_v)
        if spmem_table:
            r0 = s * rstage
            pltpu.sync_copy(table_hbm.at[pl.ds(r0, rstage)],
                            tbl.at[pl.ds(r0, rstage)])
            if rtail:
                @pl.when(s == _NS - 1)
                def _():
                    pltpu.sync_copy(
                        table_hbm.at[pl.ds(rstage * _NS, rtail)],
                        tbl.at[pl.ds(rstage * _NS, rtail)])
            plsc.subcore_barrier()
        for b in range(min(nbuf, nsteps)):
            pltpu.async_copy(tbl.at[idx_v.at[b]], bufs.at[b], gsem)

        def _step(step, b):
            pltpu.make_async_copy(
                tbl.at[idx_v.at[0]], bufs.at[b], gsem).wait()
            pltpu.async_copy(
                bufs.at[b], out_hbm.at[pl.ds(base + step * k, k)], ssem)
            pltpu.make_async_copy(
                bufs.at[b], out_hbm.at[pl.ds(base, k)], ssem).wait()
            nstep = step + nbuf

            @pl.when(nstep < nsteps)
            def _():
                pltpu.async_copy(tbl.at[idx_v.at[nstep]], bufs.at[b], gsem)

        def group(gi, carry):
            for b in range(nbuf):
                _step(gi * nbuf + b, b)
            return carry

        lax.fori_loop(0, ngroups, group, 0)
        for t in range(ntail):
            _step(ngroups * nbuf + t, t)

    return gather_kernel


def _make_sc_scatter(n_edges, n_nodes, hdim, k, nbuf):
    ec = n_edges // _NW
    nsteps = ec // k
    rz = n_nodes // _NS
    ngroups, ntail = divmod(nsteps, nbuf)
    assert ec % k == 0 and n_nodes % (8 * _NS) == 0
    mesh = plsc.VectorSubcoreMesh(core_axis_name="c", subcore_axis_name="s",
                                  num_cores=_NC, num_subcores=_NS)

    @functools.partial(
        pl.kernel,
        out_type=jax.ShapeDtypeStruct((_NC * n_nodes, hdim), jnp.float32),
        mesh=mesh,
        scratch_types=[
            pltpu.VMEM((nsteps, k), jnp.int32),
            pltpu.VMEM((nbuf, k, hdim), jnp.float32),
            pltpu.VMEM_SHARED((n_nodes, hdim), jnp.float32),
            pltpu.SemaphoreType.DMA,
            pltpu.SemaphoreType.DMA,
        ],
    )
    def scatter_kernel(msg_hbm, idx_hbm, zeros_hbm, out_hbm, idx_v, bufs, acc,
                       gsem, asem):
        c = lax.axis_index("c")
        s = lax.axis_index("s")
        w = c * _NS + s
        base = w * ec
        pltpu.sync_copy(zeros_hbm.at[pl.ds(s * rz, rz)], acc.at[pl.ds(s * rz, rz)])
        pltpu.sync_copy(idx_hbm.at[w], idx_v)
        plsc.subcore_barrier()
        for b in range(nbuf):
            pltpu.async_copy(
                msg_hbm.at[pl.ds(base + b * k, k)], bufs.at[b], gsem)

        def _step(step, b):
            pltpu.make_async_copy(
                msg_hbm.at[pl.ds(base, k)], bufs.at[b], gsem).wait()
            pltpu.async_copy(
                bufs.at[b], acc.at[idx_v.at[step]], asem, add=True)
            pltpu.make_async_copy(
                bufs.at[b], acc.at[idx_v.at[0]], asem).wait()
            nstep = step + nbuf

            @pl.when(nstep < nsteps)
            def _():
                pltpu.async_copy(
                    msg_hbm.at[pl.ds(base + nstep * k, k)], bufs.at[b], gsem)

        def group(gi, carry):
            for b in range(nbuf):
                _step(gi * nbuf + b, b)
            return carry

        lax.fori_loop(0, ngroups, group, 0)
        for t in range(ntail):
            _step(ngroups * nbuf + t, t)
        plsc.subcore_barrier()
        pltpu.sync_copy(acc.at[pl.ds(s * rz, rz)],
                        out_hbm.at[pl.ds(c * n_nodes + s * rz, rz)])

    return scatter_kernel


def kernel(x, edge_index, edge_attr, W_ne, b_ne, W_ee, b_ee, Wn, bn, Wg, bg,
           We, be, W1, b1, W2, b2):
    n, _ = x.shape
    e_cnt = edge_index.shape[1]
    hdim = W_ne.shape[1]
    n_layers = Wn.shape[0]
    eh = e_cnt // 2

    gk = 80
    sk = 40
    beh = 8000
    n_pad = ((n + 639) // 640) * 640
    gather_fn = _make_sc_gather(2 * eh, n, hdim, gk, nbuf=3,
                                spmem_table=True)
    scatter_fn = _make_sc_scatter(eh, n_pad, hdim, sk, nbuf=3)

    row = edge_index[0]
    col = edge_index[1]
    halves = []
    for half in range(2):
        sl = slice(half * eh, (half + 1) * eh)
        idx_h = jnp.concatenate([row[sl], col[sl]]).reshape(
            _NW, (2 * eh) // (_NW * gk), gk)
        col_h = col[sl].reshape(_NW, eh // (_NW * sk), sk)
        halves.append((idx_h, col_h, edge_attr[sl]))
    zeros = jnp.zeros((n_pad, hdim), jnp.float32)

    h = _linear(x, W_ne, b_ne)
    e_half = [(ea, W_ee, b_ee) for (_, _, ea) in halves]

    for l in range(n_layers):
        wg_x, wg_e = Wg[l, :hdim], Wg[l, hdim:]
        wn_x, wn_e = Wn[l, :hdim], Wn[l, hdim:]
        we_r, we_c, we_e = We[l, :hdim], We[l, hdim:2 * hdim], We[l, 2 * hdim:]
        gaths = [gather_fn(h, halves[0][0]), gather_fn(h, halves[1][0])]
        partials = []
        if l < n_layers - 1:
            wxi = jnp.concatenate([wg_x, we_c], axis=1)
            wxj = jnp.concatenate([wn_x, we_r], axis=1)
            wee = jnp.concatenate([wg_e, wn_e, we_e], axis=1)
            for half in range(2):
                msg, e_half[half] = _edge_layer(
                    gaths[half], e_half[half], wxi, wxj, wee, bg[l], bn[l],
                    be[l], eh, block_edges=beh)
                partials.append(scatter_fn(msg, halves[half][1], zeros))
            h = _node_update(h, partials, n_pad)
        else:
            wge = jnp.concatenate([wg_e, wn_e], axis=1)
            for half in range(2):
                msg = _edge_layer_last(
                    gaths[half], e_half[half], wg_x, wn_x, wge, bg[l], bn[l],
                    eh, block_edges=beh)
                partials.append(scatter_fn(msg, halves[half][1], zeros))
            out = _final(h, partials, W1, b1, W2, b2, n_pad)
    return out
